# baseline (speedup 1.0000x reference)
import functools

import jax
import jax.numpy as jnp
from jax import lax
from jax.experimental import pallas as pl
from jax.experimental.pallas import tpu as pltpu
from jax.experimental.pallas import tpu_sc as plsc

B, N, D = 4, 256, 64
E = N * N
NW = 32
EPW = E // NW
CH = 512
ROWS_PW = 2 * EPW * B
NCHUNK = ROWS_PW // CH
CPB = 2 * EPW // CH
NB = 1
NG = NCHUNK // NB
NP = NG // 2


def _sc_gather(hv_flat, v1, v2):
    mesh = plsc.VectorSubcoreMesh(core_axis_name="c", subcore_axis_name="s")

    @functools.partial(
        pl.kernel,
        mesh=mesh,
        compiler_params=pltpu.CompilerParams(
            needs_layout_passes=False, use_tc_tiling_on_sc=False),
        out_type=jax.ShapeDtypeStruct((B * E * 2, D), jnp.float32),
        scratch_types=[
            pltpu.VMEM((EPW,), jnp.int32),
            pltpu.VMEM((EPW,), jnp.int32),
            pltpu.VMEM((NCHUNK * CH,), jnp.int32),
            pltpu.VMEM_SHARED((B * N, D), jnp.float32),
            pltpu.VMEM((2, NB, CH, D), jnp.float32),
            pltpu.SemaphoreType.DMA,
            pltpu.SemaphoreType.DMA,
            pltpu.SemaphoreType.DMA,
        ],
    )
    def k(hv_hbm, v1_hbm, v2_hbm, out_hbm,
          v1_v, v2_v, idx_v, hv_v, bufs, gsem, ssem_a, ssem_b):
        wid = lax.axis_index("s") * 2 + lax.axis_index("c")
        ebase = wid * EPW

        @pl.when(lax.axis_index("s") == 0)
        def _():
            pltpu.sync_copy(hv_hbm, hv_v)

        pltpu.sync_copy(v1_hbm.at[pl.ds(ebase, EPW)], v1_v)
        pltpu.sync_copy(v2_hbm.at[pl.ds(ebase, EPW)], v2_v)

        iota = lax.iota(jnp.int32, 16)

        def build(j, _):
            a = v1_v[pl.ds(j * 16, 16)]
            bvec = v2_v[pl.ds(j * 16, 16)]
            pos_a = j * 32 + 2 * iota
            for bb in range(B):
                plsc.store_scatter(idx_v, [pos_a + bb * (2 * EPW)], a + N * bb)
                plsc.store_scatter(
                    idx_v, [pos_a + bb * (2 * EPW) + 1], bvec + N * bb)
            return 0

        lax.fori_loop(0, EPW // 16, build, 0)

        def out_base(c):
            bb = c // CPB
            lc = c - bb * CPB
            return bb * (2 * E) + wid * (2 * EPW) + lc * CH

        def g_copy(c, dset, s):
            return pltpu.make_async_copy(
                hv_v.at[idx_v.at[pl.ds(c * CH, CH)]],
                bufs.at[dset, s], gsem)

        def s_copy(c, dset, s, sem):
            return pltpu.make_async_copy(
                bufs.at[dset, s], out_hbm.at[pl.ds(out_base(c), CH)], sem)

        plsc.subcore_barrier()
        for s in range(NB):
            g_copy(s, 0, s).start()

        def step(p, _):
            ca = (2 * p) * NB
            cb = ca + NB
            for s in range(NB):
                g_copy(ca + s, 0, s).wait()
            for s in range(NB):
                s_copy(ca + s, 0, s, ssem_a).start()

            @pl.when(p > 0)
            def _():
                for s in range(NB):
                    s_copy(cb - 2 * NB + s, 1, s, ssem_b).wait()

            for s in range(NB):
                g_copy(cb + s, 1, s).start()
            for s in range(NB):
                g_copy(cb + s, 1, s).wait()
            for s in range(NB):
                s_copy(cb + s, 1, s, ssem_b).start()
            for s in range(NB):
                s_copy(ca + s, 0, s, ssem_a).wait()

            @pl.when(p < NP - 1)
            def _():
                for s in range(NB):
                    g_copy(cb + NB + s, 0, s).start()

            return 0

        lax.fori_loop(0, NP, step, 0)
        for s in range(NB):
            s_copy(NCHUNK - NB + s, 1, s, ssem_b).wait()

    return k(hv_flat, v1, v2)


def kernel(hv, v1s_idx, v2s_idx):
    hv_flat = hv.reshape(B * N, D)
    v1 = v1s_idx.astype(jnp.int32)
    v2 = v2s_idx.astype(jnp.int32)
    out = _sc_gather(hv_flat, v1, v2)
    return out.reshape(B, N, N, 2 * D)


# speedup vs baseline: 1.0050x; 1.0050x over previous
import functools

import jax
import jax.numpy as jnp
from jax import lax
from jax.experimental import pallas as pl
from jax.experimental.pallas import tpu as pltpu
from jax.experimental.pallas import tpu_sc as plsc

B, N, D = 4, 256, 64
E = N * N
NW = 32
EPW = E // NW
CH = 256
ROWS_PW = 2 * EPW * B
NCHUNK = ROWS_PW // CH
CPB = 2 * EPW // CH
NB = 2
NG = NCHUNK // NB
NP = NG // 2


def _sc_gather(hv_flat, v1, v2):
    mesh = plsc.VectorSubcoreMesh(core_axis_name="c", subcore_axis_name="s")

    @functools.partial(
        pl.kernel,
        mesh=mesh,
        compiler_params=pltpu.CompilerParams(
            needs_layout_passes=False, use_tc_tiling_on_sc=False),
        out_type=jax.ShapeDtypeStruct((B * E * 2, D), jnp.float32),
        scratch_types=[
            pltpu.VMEM((EPW,), jnp.int32),
            pltpu.VMEM((EPW,), jnp.int32),
            pltpu.VMEM((NCHUNK * CH,), jnp.int32),
            pltpu.VMEM_SHARED((B * N, D), jnp.float32),
            pltpu.VMEM((2, NB, CH, D), jnp.float32),
            pltpu.SemaphoreType.DMA,
            pltpu.SemaphoreType.DMA,
            pltpu.SemaphoreType.DMA,
        ],
    )
    def k(hv_hbm, v1_hbm, v2_hbm, out_hbm,
          v1_v, v2_v, idx_v, hv_v, bufs, gsem, ssem_a, ssem_b):
        wid = lax.axis_index("s") * 2 + lax.axis_index("c")
        ebase = wid * EPW

        @pl.when(lax.axis_index("s") == 0)
        def _():
            pltpu.sync_copy(hv_hbm, hv_v)

        pltpu.sync_copy(v1_hbm.at[pl.ds(ebase, EPW)], v1_v)
        pltpu.sync_copy(v2_hbm.at[pl.ds(ebase, EPW)], v2_v)

        iota = lax.iota(jnp.int32, 16)

        def build(j, _):
            a = v1_v[pl.ds(j * 16, 16)]
            bvec = v2_v[pl.ds(j * 16, 16)]
            pos_a = j * 32 + 2 * iota
            plsc.store_scatter(idx_v, [pos_a], a)
            plsc.store_scatter(idx_v, [pos_a + 1], bvec)
            return 0

        lax.fori_loop(0, EPW // 16, build, 0)

        def build_rest(j, _):
            a = v1_v[pl.ds(j * 16, 16)]
            bvec = v2_v[pl.ds(j * 16, 16)]
            pos_a = j * 32 + 2 * iota
            for bb in range(1, B):
                plsc.store_scatter(idx_v, [pos_a + bb * (2 * EPW)], a + N * bb)
                plsc.store_scatter(
                    idx_v, [pos_a + bb * (2 * EPW) + 1], bvec + N * bb)
            return 0

        def out_base(c):
            bb = c // CPB
            lc = c - bb * CPB
            return bb * (2 * E) + wid * (2 * EPW) + lc * CH

        def g_copy(c, dset, s):
            return pltpu.make_async_copy(
                hv_v.at[idx_v.at[pl.ds(c * CH, CH)]],
                bufs.at[dset, s], gsem)

        def s_copy(c, dset, s, sem):
            return pltpu.make_async_copy(
                bufs.at[dset, s], out_hbm.at[pl.ds(out_base(c), CH)], sem)

        plsc.subcore_barrier()
        for s in range(NB):
            g_copy(s, 0, s).start()
        lax.fori_loop(0, EPW // 16, build_rest, 0)

        def step(p, _):
            ca = (2 * p) * NB
            cb = ca + NB
            for s in range(NB):
                g_copy(ca + s, 0, s).wait()
            for s in range(NB):
                s_copy(ca + s, 0, s, ssem_a).start()

            @pl.when(p > 0)
            def _():
                for s in range(NB):
                    s_copy(cb - 2 * NB + s, 1, s, ssem_b).wait()

            for s in range(NB):
                g_copy(cb + s, 1, s).start()
            for s in range(NB):
                g_copy(cb + s, 1, s).wait()
            for s in range(NB):
                s_copy(cb + s, 1, s, ssem_b).start()
            for s in range(NB):
                s_copy(ca + s, 0, s, ssem_a).wait()

            @pl.when(p < NP - 1)
            def _():
                for s in range(NB):
                    g_copy(cb + NB + s, 0, s).start()

            return 0

        lax.fori_loop(0, NP, step, 0)
        for s in range(NB):
            s_copy(NCHUNK - NB + s, 1, s, ssem_b).wait()

    return k(hv_flat, v1, v2)


def kernel(hv, v1s_idx, v2s_idx):
    hv_flat = hv.reshape(B * N, D)
    v1 = v1s_idx.astype(jnp.int32)
    v2 = v2s_idx.astype(jnp.int32)
    out = _sc_gather(hv_flat, v1, v2)
    return out.reshape(B, N, N, 2 * D)


# speedup vs baseline: 1.0254x; 1.0203x over previous
import functools

import jax
import jax.numpy as jnp
from jax import lax
from jax.experimental import pallas as pl
from jax.experimental.pallas import tpu as pltpu
from jax.experimental.pallas import tpu_sc as plsc

B, N, D = 4, 256, 64
E = N * N
NW = 32
EPW = E // NW
CH = 256
ROWS_PW = 2 * EPW * B
NCHUNK = ROWS_PW // CH
CPB = 2 * EPW // CH
NB = 2
NG = NCHUNK // NB
NP = NG // 2


def _sc_gather(hv_flat, v1, v2):
    mesh = plsc.VectorSubcoreMesh(core_axis_name="c", subcore_axis_name="s")

    @functools.partial(
        pl.kernel,
        mesh=mesh,
        compiler_params=pltpu.CompilerParams(
            needs_layout_passes=False, use_tc_tiling_on_sc=False),
        out_type=jax.ShapeDtypeStruct((B * E * 2, D), jnp.float32),
        scratch_types=[
            pltpu.VMEM((EPW,), jnp.int32),
            pltpu.VMEM((EPW,), jnp.int32),
            pltpu.VMEM((NCHUNK * CH,), jnp.int32),
            pltpu.VMEM_SHARED((B * N, D), jnp.float32),
            pltpu.VMEM((2, NB, CH, D), jnp.float32),
            pltpu.SemaphoreType.DMA,
            pltpu.SemaphoreType.DMA,
            pltpu.SemaphoreType.DMA,
            pltpu.SemaphoreType.DMA,
        ],
    )
    def k(hv_hbm, v1_hbm, v2_hbm, out_hbm,
          v1_v, v2_v, idx_v, hv_v, bufs, gsem, ssem_a, ssem_b, psem):
        wid = lax.axis_index("s") * 2 + lax.axis_index("c")
        ebase = wid * EPW
        sub0 = lax.axis_index("s") == 0

        @pl.when(sub0)
        def _():
            pltpu.async_copy(hv_hbm, hv_v, psem)

        v1c = pltpu.make_async_copy(v1_hbm.at[pl.ds(ebase, EPW)], v1_v, gsem)
        v2c = pltpu.make_async_copy(v2_hbm.at[pl.ds(ebase, EPW)], v2_v, gsem)
        v1c.start()
        v2c.start()
        v1c.wait()
        v2c.wait()

        iota = lax.iota(jnp.int32, 16)

        def build(j, _):
            a = v1_v[pl.ds(j * 16, 16)]
            bvec = v2_v[pl.ds(j * 16, 16)]
            pos_a = j * 32 + 2 * iota
            plsc.store_scatter(idx_v, [pos_a], a)
            plsc.store_scatter(idx_v, [pos_a + 1], bvec)
            return 0

        lax.fori_loop(0, EPW // 16, build, 0)

        def build_rest(j, _):
            a = v1_v[pl.ds(j * 16, 16)]
            bvec = v2_v[pl.ds(j * 16, 16)]
            pos_a = j * 32 + 2 * iota
            for bb in range(1, B):
                plsc.store_scatter(idx_v, [pos_a + bb * (2 * EPW)], a + N * bb)
                plsc.store_scatter(
                    idx_v, [pos_a + bb * (2 * EPW) + 1], bvec + N * bb)
            return 0

        def out_base(c):
            bb = c // CPB
            lc = c - bb * CPB
            return bb * (2 * E) + wid * (2 * EPW) + lc * CH

        def g_copy(c, dset, s):
            return pltpu.make_async_copy(
                hv_v.at[idx_v.at[pl.ds(c * CH, CH)]],
                bufs.at[dset, s], gsem)

        def s_copy(c, dset, s, sem):
            return pltpu.make_async_copy(
                bufs.at[dset, s], out_hbm.at[pl.ds(out_base(c), CH)], sem)

        @pl.when(sub0)
        def _():
            pltpu.make_async_copy(hv_hbm, hv_v, psem).wait()

        plsc.subcore_barrier()
        for s in range(NB):
            g_copy(s, 0, s).start()
        lax.fori_loop(0, EPW // 16, build_rest, 0)

        def step(p, _):
            ca = (2 * p) * NB
            cb = ca + NB
            for s in range(NB):
                g_copy(ca + s, 0, s).wait()
            for s in range(NB):
                s_copy(ca + s, 0, s, ssem_a).start()

            @pl.when(p > 0)
            def _():
                for s in range(NB):
                    s_copy(cb - 2 * NB + s, 1, s, ssem_b).wait()

            for s in range(NB):
                g_copy(cb + s, 1, s).start()
            for s in range(NB):
                g_copy(cb + s, 1, s).wait()
            for s in range(NB):
                s_copy(cb + s, 1, s, ssem_b).start()
            for s in range(NB):
                s_copy(ca + s, 0, s, ssem_a).wait()

            @pl.when(p < NP - 1)
            def _():
                for s in range(NB):
                    g_copy(cb + NB + s, 0, s).start()

            return 0

        lax.fori_loop(0, NP, step, 0)
        for s in range(NB):
            s_copy(NCHUNK - NB + s, 1, s, ssem_b).wait()

    return k(hv_flat, v1, v2)


def kernel(hv, v1s_idx, v2s_idx):
    hv_flat = hv.reshape(B * N, D)
    v1 = v1s_idx.astype(jnp.int32)
    v2 = v2s_idx.astype(jnp.int32)
    out = _sc_gather(hv_flat, v1, v2)
    return out.reshape(B, N, N, 2 * D)
